# baseline (speedup 1.0000x reference)
import functools

import jax
import jax.numpy as jnp
from jax import lax
from jax.experimental import pallas as pl
from jax.experimental.pallas import tpu as pltpu
from jax.experimental.pallas import tpu_sc as plsc

_WAVE = 16


def kernel(x, weight):
    (B,) = x.shape
    V, D = weight.shape

    info = plsc.get_sparse_core_info()
    NC, NS = info.num_cores, info.num_subcores
    NW = NC * NS
    b_per_w = B // NW
    n_wave = b_per_w // _WAVE

    xi = x.astype(jnp.int32)

    mesh = plsc.VectorSubcoreMesh(core_axis_name="c", subcore_axis_name="s")

    @functools.partial(
        pl.kernel,
        mesh=mesh,
        out_type=jax.ShapeDtypeStruct((B, D), jnp.float32),
        scratch_types=[
            pltpu.VMEM((b_per_w,), jnp.int32),
            pltpu.VMEM((b_per_w, D), jnp.float32),
            pltpu.SemaphoreType.DMA,
        ],
        compiler_params=pltpu.CompilerParams(
            skip_device_barrier=True,
            disable_bounds_checks=True,
            disable_semaphore_checks=True,
        ),
    )
    def emb(x_hbm, w_hbm, out_hbm, x_v, rows_v, sem):
        wid = lax.axis_index("s") * NC + lax.axis_index("c")
        base = wid * b_per_w
        pltpu.sync_copy(x_hbm.at[pl.ds(base, b_per_w)], x_v)

        def wave(wv, _):
            xv = x_v[pl.ds(wv * _WAVE, _WAVE)]
            copies = []
            for i in range(_WAVE):
                e = wv * _WAVE + i
                t = xv[i]
                c = pltpu.make_async_copy(
                    w_hbm.at[t], rows_v.at[e], sem
                )
                c.start()
                copies.append(c)
            for c in copies:
                c.wait()
            return 0

        lax.fori_loop(0, n_wave, wave, 0)
        pltpu.sync_copy(rows_v, out_hbm.at[pl.ds(base, b_per_w)])

    return emb(xi, weight)


# speedup vs baseline: 2.2558x; 2.2558x over previous
import functools

import jax
import jax.numpy as jnp
from jax import lax
from jax.experimental import pallas as pl
from jax.experimental.pallas import tpu as pltpu
from jax.experimental.pallas import tpu_sc as plsc

_L = 16
_W = 8


def kernel(x, weight):
    (B,) = x.shape
    V, D = weight.shape

    info = plsc.get_sparse_core_info()
    NC, NS = info.num_cores, info.num_subcores
    NW = NC * NS
    b_per_w = B // NW
    n_iter = b_per_w // (2 * _W)

    xi = x.astype(jnp.int32)
    wt = weight.T

    mesh = plsc.VectorSubcoreMesh(core_axis_name="c", subcore_axis_name="s")

    @functools.partial(
        pl.kernel,
        mesh=mesh,
        out_type=jax.ShapeDtypeStruct((NW, b_per_w * D), jnp.float32),
        scratch_types=[
            pltpu.VMEM((b_per_w,), jnp.int32),
            pltpu.VMEM((2, _W, D, 128), jnp.float32),
            pltpu.VMEM((b_per_w * D,), jnp.float32),
            pltpu.SemaphoreType.DMA,
            pltpu.SemaphoreType.DMA,
        ],
        compiler_params=pltpu.CompilerParams(
            needs_layout_passes=False,
            disable_bounds_checks=True,
        ),
    )
    def emb(x_hbm, w_hbm, out_hbm, x_v, blk_v, out_v, sem_a, sem_b):
        wid = lax.axis_index("s") * NC + lax.axis_index("c")
        base = wid * b_per_w
        pltpu.sync_copy(x_hbm.at[pl.ds(base, b_per_w)], x_v)
        sems = (sem_a, sem_b)
        iota16 = jax.lax.iota(jnp.int32, _L)

        def fire_one(t, buf, i):
            blk = (t >> 7) << 7
            pltpu.make_async_copy(
                w_hbm.at[:, pl.ds(pl.multiple_of(blk, 128), 128)],
                blk_v.at[buf, i],
                sems[buf],
            ).start()

        def drain_one(buf, i):
            pltpu.make_async_copy(
                w_hbm.at[:, pl.ds(0, 128)],
                blk_v.at[buf, i],
                sems[buf],
            ).wait()

        def extract_one(t, buf, i, e):
            c = t & 127
            col16 = jnp.full((_L,), 0, jnp.int32) + c
            v_lo = plsc.load_gather(blk_v.at[buf, i], [iota16, col16])
            v_hi = plsc.load_gather(blk_v.at[buf, i], [iota16 + _L, col16])
            out_v[pl.ds(e * D, _L)] = v_lo
            out_v[pl.ds(e * D + _L, _L)] = v_hi

        xv0 = x_v[pl.ds(0, _L)]
        for i in range(_W):
            fire_one(xv0[i], 0, i)

        def body(k, _):
            xv = x_v[pl.ds(k * 2 * _W, _L)]
            ts = [xv[i] for i in range(_L)]
            for i in range(_W):
                fire_one(ts[_W + i], 1, i)
            for i in range(_W):
                drain_one(0, i)
            for i in range(_W):
                extract_one(ts[i], 0, i, k * 2 * _W + i)

            @pl.when(k + 1 < n_iter)
            def _():
                xvn = x_v[pl.ds((k + 1) * 2 * _W, _L)]
                for i in range(_W):
                    fire_one(xvn[i], 0, i)

            for i in range(_W):
                drain_one(1, i)
            for i in range(_W):
                extract_one(ts[_W + i], 1, i, k * 2 * _W + _W + i)
            return 0

        lax.fori_loop(0, n_iter, body, 0)
        pltpu.sync_copy(out_v, out_hbm.at[wid])

    return emb(xi, wt).reshape(B, D)


# speedup vs baseline: 2.2603x; 1.0020x over previous
import functools

import jax
import jax.numpy as jnp
from jax import lax
from jax.experimental import pallas as pl
from jax.experimental.pallas import tpu as pltpu
from jax.experimental.pallas import tpu_sc as plsc

_L = 16
_W = 8
_P = 3


def kernel(x, weight):
    (B,) = x.shape
    V, D = weight.shape

    info = plsc.get_sparse_core_info()
    NC, NS = info.num_cores, info.num_subcores
    NW = NC * NS
    b_per_w = B // NW
    n_wave = b_per_w // _W

    xi = x.astype(jnp.int32)
    wt = weight.T

    mesh = plsc.VectorSubcoreMesh(core_axis_name="c", subcore_axis_name="s")

    @functools.partial(
        pl.kernel,
        mesh=mesh,
        out_type=jax.ShapeDtypeStruct((NW, b_per_w * D), jnp.float32),
        scratch_types=[
            pltpu.VMEM((b_per_w,), jnp.int32),
            pltpu.VMEM((_P, _W, D, 128), jnp.float32),
            pltpu.VMEM((b_per_w * D,), jnp.float32),
            pltpu.SemaphoreType.DMA,
            pltpu.SemaphoreType.DMA,
            pltpu.SemaphoreType.DMA,
        ],
        compiler_params=pltpu.CompilerParams(
            needs_layout_passes=False,
            disable_bounds_checks=True,
        ),
    )
    def emb(x_hbm, w_hbm, out_hbm, x_v, blk_v, out_v, sem_a, sem_b, sem_c):
        wid = lax.axis_index("s") * NC + lax.axis_index("c")
        base = wid * b_per_w
        pltpu.sync_copy(x_hbm.at[pl.ds(base, b_per_w)], x_v)
        sems = (sem_a, sem_b, sem_c)
        iota16 = jax.lax.iota(jnp.int32, _L)

        ts = []
        for g in range(b_per_w // _L):
            xv = x_v[pl.ds(g * _L, _L)]
            ts.extend(xv[i] for i in range(_L))

        def fire(w):
            p = w % _P
            for i in range(_W):
                t = ts[w * _W + i]
                blk = (t >> 7) << 7
                pltpu.make_async_copy(
                    w_hbm.at[:, pl.ds(pl.multiple_of(blk, 128), 128)],
                    blk_v.at[p, i],
                    sems[p],
                ).start()

        def drain_extract(w):
            p = w % _P
            for i in range(_W):
                pltpu.make_async_copy(
                    w_hbm.at[:, pl.ds(0, 128)],
                    blk_v.at[p, i],
                    sems[p],
                ).wait()
            for i in range(_W):
                t = ts[w * _W + i]
                col16 = jnp.full((_L,), 0, jnp.int32) + (t & 127)
                v_lo = plsc.load_gather(blk_v.at[p, i], [iota16, col16])
                v_hi = plsc.load_gather(blk_v.at[p, i], [iota16 + _L, col16])
                e = w * _W + i
                out_v[pl.ds(e * D, _L)] = v_lo
                out_v[pl.ds(e * D + _L, _L)] = v_hi

        for w in range(_P - 1):
            fire(w)
        for w in range(n_wave):
            if w + _P - 1 < n_wave:
                fire(w + _P - 1)
            drain_extract(w)

        pltpu.sync_copy(out_v, out_hbm.at[wid])

    return emb(xi, wt).reshape(B, D)


# speedup vs baseline: 2.2654x; 1.0022x over previous
import functools

import jax
import jax.numpy as jnp
from jax import lax
from jax.experimental import pallas as pl
from jax.experimental.pallas import tpu as pltpu
from jax.experimental.pallas import tpu_sc as plsc

_L = 16
_W = 8


def kernel(x, weight):
    (B,) = x.shape
    V, D = weight.shape

    info = plsc.get_sparse_core_info()
    NC, NS = info.num_cores, info.num_subcores
    NW = NC * NS
    b_per_w = B // NW
    n_iter = b_per_w // (2 * _W)

    xi = x.astype(jnp.int32)
    wt = weight.T

    mesh = plsc.VectorSubcoreMesh(core_axis_name="c", subcore_axis_name="s")

    @functools.partial(
        pl.kernel,
        mesh=mesh,
        out_type=jax.ShapeDtypeStruct((NW, b_per_w * D), jnp.float32),
        scratch_types=[
            pltpu.VMEM((b_per_w,), jnp.int32),
            pltpu.VMEM((2, _W, D, 128), jnp.float32),
            pltpu.VMEM((b_per_w * D,), jnp.float32),
            pltpu.SemaphoreType.DMA,
            pltpu.SemaphoreType.DMA,
        ],
        compiler_params=pltpu.CompilerParams(
            needs_layout_passes=False,
            disable_bounds_checks=True,
        ),
    )
    def emb(x_hbm, w_hbm, out_hbm, x_v, blk_v, out_v, sem_a, sem_b):
        wid = lax.axis_index("s") * NC + lax.axis_index("c")
        base = wid * b_per_w
        pltpu.sync_copy(x_hbm.at[pl.ds(base, b_per_w)], x_v)
        sems = (sem_a, sem_b)
        iota16 = jax.lax.iota(jnp.int32, _L)

        def fire_one(t, buf, i):
            blk = (t >> 7) << 7
            pltpu.make_async_copy(
                w_hbm.at[:, pl.ds(pl.multiple_of(blk, 128), 128)],
                blk_v.at[buf, i],
                sems[buf],
            ).start()

        def drain_one(buf, i):
            pltpu.make_async_copy(
                w_hbm.at[:, pl.ds(0, 128)],
                blk_v.at[buf, i],
                sems[buf],
            ).wait()

        def extract_one(t, buf, i, e):
            c = t & 127
            col16 = jnp.full((_L,), 0, jnp.int32) + c
            v_lo = plsc.load_gather(blk_v.at[buf, i], [iota16, col16])
            v_hi = plsc.load_gather(blk_v.at[buf, i], [iota16 + _L, col16])
            out_v[pl.ds(e * D, _L)] = v_lo
            out_v[pl.ds(e * D + _L, _L)] = v_hi

        xv0 = x_v[pl.ds(0, _L)]
        for i in range(_W):
            fire_one(xv0[i], 0, i)

        def body(k, _):
            xv = x_v[pl.ds(k * 2 * _W, _L)]
            ts = [xv[i] for i in range(_L)]
            for i in range(_W):
                fire_one(ts[_W + i], 1, i)
            for i in range(_W):
                drain_one(0, i)
            for i in range(_W):
                extract_one(ts[i], 0, i, k * 2 * _W + i)

            @pl.when(k + 1 < n_iter)
            def _():
                xvn = x_v[pl.ds((k + 1) * 2 * _W, _L)]
                for i in range(_W):
                    fire_one(xvn[i], 0, i)

            for i in range(_W):
                drain_one(1, i)
            for i in range(_W):
                extract_one(ts[_W + i], 1, i, k * 2 * _W + _W + i)
            return 0

        lax.fori_loop(0, n_iter, body, 0)
        pltpu.sync_copy(out_v, out_hbm.at[wid])

    return emb(xi, wt).reshape(B, D)


# speedup vs baseline: 2.2708x; 1.0024x over previous
import functools

import jax
import jax.numpy as jnp
from jax import lax
from jax.experimental import pallas as pl
from jax.experimental.pallas import tpu as pltpu
from jax.experimental.pallas import tpu_sc as plsc

_L = 16
_W = 8


def kernel(x, weight):
    (B,) = x.shape
    V, D = weight.shape

    info = plsc.get_sparse_core_info()
    NC, NS = info.num_cores, info.num_subcores
    NW = NC * NS
    b_per_w = B // NW
    n_iter = b_per_w // (2 * _W)

    xi = x.astype(jnp.int32)
    wt = weight.T

    mesh = plsc.VectorSubcoreMesh(core_axis_name="c", subcore_axis_name="s")

    @functools.partial(
        pl.kernel,
        mesh=mesh,
        out_type=jax.ShapeDtypeStruct((NW, b_per_w * D), jnp.float32),
        scratch_types=[
            pltpu.VMEM((b_per_w,), jnp.int32),
            pltpu.VMEM((2, _W, D, 128), jnp.float32),
            pltpu.VMEM((b_per_w * D,), jnp.float32),
            pltpu.SemaphoreType.DMA,
            pltpu.SemaphoreType.DMA,
        ],
        compiler_params=pltpu.CompilerParams(
            needs_layout_passes=False,
            disable_bounds_checks=True,
        ),
    )
    def emb(x_hbm, w_hbm, out_hbm, x_v, blk_v, out_v, sem_a, sem_b):
        wid = lax.axis_index("s") * NC + lax.axis_index("c")
        base = wid * b_per_w
        pltpu.sync_copy(x_hbm.at[pl.ds(base, b_per_w)], x_v)
        sems = (sem_a, sem_b)
        iota16 = jax.lax.iota(jnp.int32, _L)

        def fire_one(t, buf, i):
            blk = (t >> 7) << 7
            for b in range(D // 8):
                pltpu.make_async_copy(
                    w_hbm.at[pl.ds(b * 8, 8), pl.ds(pl.multiple_of(blk, 128), 128)],
                    blk_v.at[buf, i, pl.ds(b * 8, 8)],
                    sems[buf],
                ).start()

        def drain_one(buf, i):
            for b in range(D // 8):
                pltpu.make_async_copy(
                    w_hbm.at[pl.ds(b * 8, 8), pl.ds(0, 128)],
                    blk_v.at[buf, i, pl.ds(b * 8, 8)],
                    sems[buf],
                ).wait()

        def extract_one(t, buf, i, e):
            c = t & 127
            col16 = jnp.full((_L,), 0, jnp.int32) + c
            v_lo = plsc.load_gather(blk_v.at[buf, i], [iota16, col16])
            v_hi = plsc.load_gather(blk_v.at[buf, i], [iota16 + _L, col16])
            out_v[pl.ds(e * D, _L)] = v_lo
            out_v[pl.ds(e * D + _L, _L)] = v_hi

        xv0 = x_v[pl.ds(0, _L)]
        for i in range(_W):
            fire_one(xv0[i], 0, i)

        def body(k, _):
            xv = x_v[pl.ds(k * 2 * _W, _L)]
            ts = [xv[i] for i in range(_L)]
            for i in range(_W):
                fire_one(ts[_W + i], 1, i)
            for i in range(_W):
                drain_one(0, i)
            for i in range(_W):
                extract_one(ts[i], 0, i, k * 2 * _W + i)

            @pl.when(k + 1 < n_iter)
            def _():
                xvn = x_v[pl.ds((k + 1) * 2 * _W, _L)]
                for i in range(_W):
                    fire_one(xvn[i], 0, i)

            for i in range(_W):
                drain_one(1, i)
            for i in range(_W):
                extract_one(ts[_W + i], 1, i, k * 2 * _W + _W + i)
            return 0

        lax.fori_loop(0, n_iter, body, 0)
        pltpu.sync_copy(out_v, out_hbm.at[wid])

    return emb(xi, wt).reshape(B, D)
